# Initial kernel scaffold; baseline (speedup 1.0000x reference)
import jax
import jax.numpy as jnp
from jax.experimental import pallas as pl


def kernel(t, x, edge_index, W1, b1, W2, b2):
    raise NotImplementedError("write your pallas kernel here")


# speedup vs baseline: 19.3688x; 19.3688x over previous
import functools

import jax
import jax.numpy as jnp
from jax import lax
from jax.experimental import pallas as pl
from jax.experimental.pallas import tpu as pltpu
from jax.experimental.pallas import tpu_sc as plsc

N = 10000
D = 256
H = 5
E = 160000

HP = 8
NPAD = 10240
NC, NS = 2, 16
NW = NC * NS
CHUNK = 128
EC = 5120
NCH = EC // CHUNK
EPAD = EC * NW
PADIDX = NPAD - 8
ROWS_PER_TILE = NPAD // NS


def _sc_propagate_body(table_hbm, src_hbm, dst_hbm, init_hbm, out_hbm,
                       idx_s_v, idx_d_v, rows_v, stage_v, acc_sh, sem):
    cid = lax.axis_index("c")
    sid = lax.axis_index("s")
    wid = cid * NS + sid
    r0 = sid * ROWS_PER_TILE

    pltpu.sync_copy(src_hbm.at[wid], idx_s_v)
    pltpu.sync_copy(dst_hbm.at[wid], idx_d_v)

    pltpu.sync_copy(init_hbm.at[cid, pl.ds(r0, ROWS_PER_TILE)], stage_v)
    pltpu.sync_copy(stage_v, acc_sh.at[pl.ds(r0, ROWS_PER_TILE)])
    plsc.subcore_barrier()

    def chunk_body(j, carry):
        pltpu.async_copy(table_hbm.at[idx_s_v.at[j]], rows_v, sem).wait()
        pltpu.sync_copy(rows_v, acc_sh.at[idx_d_v.at[j]], add=True)
        return carry

    lax.fori_loop(0, NCH, chunk_body, 0)
    plsc.subcore_barrier()

    pltpu.sync_copy(acc_sh.at[pl.ds(r0, ROWS_PER_TILE)], stage_v)
    pltpu.sync_copy(stage_v, out_hbm.at[cid, pl.ds(r0, ROWS_PER_TILE)])


_sc_propagate = pl.kernel(
    _sc_propagate_body,
    out_type=jax.ShapeDtypeStruct((NC, NPAD, HP), jnp.float32),
    mesh=plsc.VectorSubcoreMesh(
        core_axis_name="c", subcore_axis_name="s",
        num_cores=NC, num_subcores=NS),
    scratch_types=[
        pltpu.VMEM((NCH, CHUNK), jnp.int32),
        pltpu.VMEM((NCH, CHUNK), jnp.int32),
        pltpu.VMEM((CHUNK, HP), jnp.float32),
        pltpu.VMEM((ROWS_PER_TILE, HP), jnp.float32),
        pltpu.VMEM_SHARED((NPAD, HP), jnp.float32),
        pltpu.SemaphoreType.DMA,
    ],
    compiler_params=pltpu.CompilerParams(use_tc_tiling_on_sc=False),
)


def _mm1_body(x_ref, w_ref, o_ref):
    o_ref[...] = jnp.dot(x_ref[...], w_ref[...],
                         preferred_element_type=jnp.float32)


def _prep1_body(deg_ref, h1_ref, dinv_ref, g1_ref):
    deg = deg_ref[0] + deg_ref[1]
    dinv = jnp.where(deg > 0.5, lax.rsqrt(jnp.maximum(deg, 1.0)), 0.0)
    dinv_ref[...] = dinv
    g1_ref[...] = dinv * h1_ref[...]


def _mid_body(acc_ref, dinv_ref, b1_ref, g2_ref):
    dinv = dinv_ref[...]
    p = dinv * (acc_ref[0] + acc_ref[1]) + b1_ref[...]
    g2_ref[...] = dinv * jnp.maximum(p, 0.0)


def _final_body(acc_ref, dinv_ref, w2_ref, b2_ref, o_ref):
    p2 = dinv_ref[...] * (acc_ref[0] + acc_ref[1])
    o_ref[...] = jnp.dot(p2, w2_ref[...],
                         preferred_element_type=jnp.float32) + b2_ref[...]


_RB = 1000


def kernel(t, x, edge_index, W1, b1, W2, b2):
    del t
    f32 = jnp.float32

    src = edge_index[0].astype(jnp.int32)
    dst = edge_index[1].astype(jnp.int32)
    padv = jnp.full((EPAD - E,), PADIDX, jnp.int32)
    src_t = jnp.concatenate([src, padv]).reshape(NW, NCH, CHUNK)
    dst_t = jnp.concatenate([dst, padv]).reshape(NW, NCH, CHUNK)

    W1p = jnp.zeros((D, HP), f32).at[:, :H].set(W1)
    W2p = jnp.zeros((HP, D), f32).at[:H, :].set(W2)
    b1p = jnp.zeros((1, HP), f32).at[0, :H].set(b1)
    b2r = b2.reshape(1, D)

    real_row = (lax.broadcasted_iota(jnp.int32, (NPAD, HP), 0) < N)
    ones_table = jnp.where(real_row, 1.0, 0.0).astype(f32)
    zeros_init = jnp.zeros((NPAD, HP), f32)

    def two_part_init(part0):
        return jnp.stack([part0, zeros_init])

    h1 = pl.pallas_call(
        _mm1_body,
        grid=(N // _RB,),
        in_specs=[pl.BlockSpec((_RB, D), lambda i: (i, 0)),
                  pl.BlockSpec((D, HP), lambda i: (0, 0))],
        out_specs=pl.BlockSpec((_RB, HP), lambda i: (i, 0)),
        out_shape=jax.ShapeDtypeStruct((N, HP), f32),
    )(x, W1p)
    h1p = jnp.zeros((NPAD, HP), f32).at[:N].set(h1)

    deg_parts = _sc_propagate(ones_table, src_t, dst_t,
                              two_part_init(ones_table))

    dinv, g1 = pl.pallas_call(
        _prep1_body,
        out_shape=[jax.ShapeDtypeStruct((NPAD, HP), f32),
                   jax.ShapeDtypeStruct((NPAD, HP), f32)],
    )(deg_parts, h1p)

    acc1_parts = _sc_propagate(g1, src_t, dst_t, two_part_init(g1))

    g2 = pl.pallas_call(
        _mid_body,
        out_shape=jax.ShapeDtypeStruct((NPAD, HP), f32),
    )(acc1_parts, dinv, b1p)

    acc2_parts = _sc_propagate(g2, src_t, dst_t, two_part_init(g2))

    out = pl.pallas_call(
        _final_body,
        grid=(N // _RB,),
        in_specs=[pl.BlockSpec((NC, _RB, HP), lambda i: (0, i, 0)),
                  pl.BlockSpec((_RB, HP), lambda i: (i, 0)),
                  pl.BlockSpec((HP, D), lambda i: (0, 0)),
                  pl.BlockSpec((1, D), lambda i: (0, 0))],
        out_specs=pl.BlockSpec((_RB, D), lambda i: (i, 0)),
        out_shape=jax.ShapeDtypeStruct((N, D), f32),
    )(acc2_parts, dinv, W2p, b2r)
    return out


# speedup vs baseline: 25.7775x; 1.3309x over previous
import functools

import jax
import jax.numpy as jnp
from jax import lax
from jax.experimental import pallas as pl
from jax.experimental.pallas import tpu as pltpu
from jax.experimental.pallas import tpu_sc as plsc

N = 10000
D = 256
H = 5
E = 160000

HP = 8
NPAD = 10240
NC, NS = 2, 16
NW = NC * NS
CHUNK = 128
EC = 5120
NCH = EC // CHUNK
EPAD = EC * NW
PADIDX = NPAD - 8
ROWS_PER_TILE = NPAD // NS


def _sc_propagate_body(with_gather, table_hbm, src_hbm, dst_hbm, init_hbm,
                       out_hbm, idx_s_v, idx_d_v, rows_v, stage_v, dummy_v,
                       acc_sh, gsem, ssem):
    cid = lax.axis_index("c")
    sid = lax.axis_index("s")
    wid = cid * NS + sid
    r0 = sid * ROWS_PER_TILE

    if with_gather:
        pltpu.sync_copy(src_hbm.at[wid], idx_s_v)
    pltpu.sync_copy(dst_hbm.at[wid], idx_d_v)

    pltpu.sync_copy(init_hbm.at[cid, pl.ds(r0, ROWS_PER_TILE)], stage_v)
    pltpu.sync_copy(stage_v, acc_sh.at[pl.ds(r0, ROWS_PER_TILE)])
    plsc.subcore_barrier()

    if with_gather:
        def fire(j, c):
            pltpu.async_copy(table_hbm.at[idx_s_v.at[j]], rows_v.at[j], gsem)
            return c
        lax.fori_loop(0, NCH, fire, 0)

        def scat(j, c):
            pltpu.make_async_copy(
                table_hbm.at[pl.ds(0, CHUNK)], rows_v.at[j], gsem).wait()
            pltpu.async_copy(
                rows_v.at[j], acc_sh.at[idx_d_v.at[j]], ssem, add=True)
            return c
        lax.fori_loop(0, NCH, scat, 0)
    else:
        pltpu.sync_copy(table_hbm.at[pl.ds(0, CHUNK)], rows_v.at[0])

        def scat_ones(j, c):
            pltpu.async_copy(
                rows_v.at[0], acc_sh.at[idx_d_v.at[j]], ssem, add=True)
            return c
        lax.fori_loop(0, NCH, scat_ones, 0)

    def drain(j, c):
        pltpu.make_async_copy(
            table_hbm.at[pl.ds(0, CHUNK)], dummy_v, ssem).wait()
        return c
    lax.fori_loop(0, NCH, drain, 0)
    plsc.subcore_barrier()

    pltpu.sync_copy(acc_sh.at[pl.ds(r0, ROWS_PER_TILE)], stage_v)
    pltpu.sync_copy(stage_v, out_hbm.at[cid, pl.ds(r0, ROWS_PER_TILE)])


def _make_sc_propagate(with_gather):
    return pl.kernel(
        functools.partial(_sc_propagate_body, with_gather),
        out_type=jax.ShapeDtypeStruct((NC, NPAD, HP), jnp.float32),
        mesh=plsc.VectorSubcoreMesh(
            core_axis_name="c", subcore_axis_name="s",
            num_cores=NC, num_subcores=NS),
        scratch_types=[
            pltpu.VMEM((NCH, CHUNK), jnp.int32),
            pltpu.VMEM((NCH, CHUNK), jnp.int32),
            pltpu.VMEM((NCH, CHUNK, HP), jnp.float32),
            pltpu.VMEM((ROWS_PER_TILE, HP), jnp.float32),
            pltpu.VMEM((CHUNK, HP), jnp.float32),
            pltpu.VMEM_SHARED((NPAD, HP), jnp.float32),
            pltpu.SemaphoreType.DMA,
            pltpu.SemaphoreType.DMA,
        ],
        compiler_params=pltpu.CompilerParams(use_tc_tiling_on_sc=False),
    )


_sc_propagate = _make_sc_propagate(True)
_sc_degree = _make_sc_propagate(False)


def _mm1_body(x_ref, w_ref, o_ref):
    o_ref[...] = jnp.dot(x_ref[...], w_ref[...],
                         preferred_element_type=jnp.float32)


def _prep1_body(deg_ref, h1_ref, dinv_ref, g1_ref):
    deg = deg_ref[0] + deg_ref[1]
    dinv = jnp.where(deg > 0.5, lax.rsqrt(jnp.maximum(deg, 1.0)), 0.0)
    dinv_ref[...] = dinv
    g1_ref[...] = dinv * h1_ref[...]


def _mid_body(acc_ref, dinv_ref, b1_ref, g2_ref):
    dinv = dinv_ref[...]
    p = dinv * (acc_ref[0] + acc_ref[1]) + b1_ref[...]
    g2_ref[...] = dinv * jnp.maximum(p, 0.0)


def _final_body(acc_ref, dinv_ref, w2_ref, b2_ref, o_ref):
    p2 = dinv_ref[...] * (acc_ref[0] + acc_ref[1])
    o_ref[...] = jnp.dot(p2, w2_ref[...],
                         preferred_element_type=jnp.float32) + b2_ref[...]


_RB = 1000


def kernel(t, x, edge_index, W1, b1, W2, b2):
    del t
    f32 = jnp.float32

    src = edge_index[0].astype(jnp.int32)
    dst = edge_index[1].astype(jnp.int32)
    padv = PADIDX + (jnp.arange(EPAD - E, dtype=jnp.int32) % 8)
    src_t = jnp.concatenate([src, padv]).reshape(NW, NCH, CHUNK)
    dst_t = jnp.concatenate([dst, padv]).reshape(NW, NCH, CHUNK)

    W1p = jnp.zeros((D, HP), f32).at[:, :H].set(W1)
    W2p = jnp.zeros((HP, D), f32).at[:H, :].set(W2)
    b1p = jnp.zeros((1, HP), f32).at[0, :H].set(b1)
    b2r = b2.reshape(1, D)

    real_row = (lax.broadcasted_iota(jnp.int32, (NPAD, HP), 0) < N)
    ones_table = jnp.where(real_row, 1.0, 0.0).astype(f32)
    zeros_init = jnp.zeros((NPAD, HP), f32)

    def two_part_init(part0):
        return jnp.stack([part0, zeros_init])

    h1 = pl.pallas_call(
        _mm1_body,
        grid=(N // _RB,),
        in_specs=[pl.BlockSpec((_RB, D), lambda i: (i, 0)),
                  pl.BlockSpec((D, HP), lambda i: (0, 0))],
        out_specs=pl.BlockSpec((_RB, HP), lambda i: (i, 0)),
        out_shape=jax.ShapeDtypeStruct((N, HP), f32),
    )(x, W1p)
    h1p = jnp.zeros((NPAD, HP), f32).at[:N].set(h1)

    deg_parts = _sc_degree(ones_table, src_t, dst_t,
                           two_part_init(ones_table))

    dinv, g1 = pl.pallas_call(
        _prep1_body,
        out_shape=[jax.ShapeDtypeStruct((NPAD, HP), f32),
                   jax.ShapeDtypeStruct((NPAD, HP), f32)],
    )(deg_parts, h1p)

    acc1_parts = _sc_propagate(g1, src_t, dst_t, two_part_init(g1))

    g2 = pl.pallas_call(
        _mid_body,
        out_shape=jax.ShapeDtypeStruct((NPAD, HP), f32),
    )(acc1_parts, dinv, b1p)

    acc2_parts = _sc_propagate(g2, src_t, dst_t, two_part_init(g2))

    out = pl.pallas_call(
        _final_body,
        grid=(N // _RB,),
        in_specs=[pl.BlockSpec((NC, _RB, HP), lambda i: (0, i, 0)),
                  pl.BlockSpec((_RB, HP), lambda i: (i, 0)),
                  pl.BlockSpec((HP, D), lambda i: (0, 0)),
                  pl.BlockSpec((1, D), lambda i: (0, 0))],
        out_specs=pl.BlockSpec((_RB, D), lambda i: (i, 0)),
        out_shape=jax.ShapeDtypeStruct((N, D), f32),
    )(acc2_parts, dinv, W2p, b2r)
    return out


# speedup vs baseline: 31.1532x; 1.2085x over previous
import functools

import jax
import jax.numpy as jnp
from jax import lax
from jax.experimental import pallas as pl
from jax.experimental.pallas import tpu as pltpu
from jax.experimental.pallas import tpu_sc as plsc

N = 10000
D = 256
H = 5
E = 160000

HP = 8
NPAD = 10240
NC, NS = 1, 16
NW = NC * NS
CHUNK = 128
EC = 10240
NCH = EC // CHUNK
EPAD = EC * NW
PADIDX = NPAD - 8
ROWS_PER_TILE = NPAD // NS


def _sc_propagate_body(with_gather, table_hbm, src_hbm, dst_hbm, init_hbm,
                       out_hbm, idx_s_v, idx_d_v, rows_v, stage_v, dummy_v,
                       acc_sh, gsem, ssem):
    cid = lax.axis_index("c")
    sid = lax.axis_index("s")
    wid = cid * NS + sid
    r0 = sid * ROWS_PER_TILE

    if with_gather:
        pltpu.sync_copy(src_hbm.at[wid], idx_s_v)
    pltpu.sync_copy(dst_hbm.at[wid], idx_d_v)

    pltpu.sync_copy(init_hbm.at[cid, pl.ds(r0, ROWS_PER_TILE)], stage_v)
    pltpu.sync_copy(stage_v, acc_sh.at[pl.ds(r0, ROWS_PER_TILE)])
    plsc.subcore_barrier()

    if with_gather:
        def fire(j, c):
            pltpu.async_copy(table_hbm.at[idx_s_v.at[j]], rows_v.at[j], gsem)
            return c
        lax.fori_loop(0, NCH, fire, 0)

        def scat(j, c):
            pltpu.make_async_copy(
                table_hbm.at[pl.ds(0, CHUNK)], rows_v.at[j], gsem).wait()
            pltpu.async_copy(
                rows_v.at[j], acc_sh.at[idx_d_v.at[j]], ssem, add=True)
            return c
        lax.fori_loop(0, NCH, scat, 0)
    else:
        pltpu.sync_copy(table_hbm.at[pl.ds(0, CHUNK)], rows_v.at[0])

        def scat_ones(j, c):
            pltpu.async_copy(
                rows_v.at[0], acc_sh.at[idx_d_v.at[j]], ssem, add=True)
            return c
        lax.fori_loop(0, NCH, scat_ones, 0)

    def drain(j, c):
        pltpu.make_async_copy(
            table_hbm.at[pl.ds(0, CHUNK)], dummy_v, ssem).wait()
        return c
    lax.fori_loop(0, NCH, drain, 0)
    plsc.subcore_barrier()

    pltpu.sync_copy(acc_sh.at[pl.ds(r0, ROWS_PER_TILE)], stage_v)
    pltpu.sync_copy(stage_v, out_hbm.at[cid, pl.ds(r0, ROWS_PER_TILE)])


def _make_sc_propagate(with_gather):
    return pl.kernel(
        functools.partial(_sc_propagate_body, with_gather),
        out_type=jax.ShapeDtypeStruct((NC, NPAD, HP), jnp.float32),
        mesh=plsc.VectorSubcoreMesh(
            core_axis_name="c", subcore_axis_name="s",
            num_cores=NC, num_subcores=NS),
        scratch_types=[
            pltpu.VMEM((NCH, CHUNK), jnp.int32),
            pltpu.VMEM((NCH, CHUNK), jnp.int32),
            pltpu.VMEM((NCH, CHUNK, HP), jnp.float32),
            pltpu.VMEM((ROWS_PER_TILE, HP), jnp.float32),
            pltpu.VMEM((CHUNK, HP), jnp.float32),
            pltpu.VMEM_SHARED((NPAD, HP), jnp.float32),
            pltpu.SemaphoreType.DMA,
            pltpu.SemaphoreType.DMA,
        ],
        compiler_params=pltpu.CompilerParams(use_tc_tiling_on_sc=False),
    )


_sc_propagate = _make_sc_propagate(True)
_sc_degree = _make_sc_propagate(False)


def _mm1_body(x_ref, w_ref, o_ref):
    o_ref[...] = jnp.dot(x_ref[...], w_ref[...],
                         preferred_element_type=jnp.float32)


def _prep1_body(deg_ref, h1_ref, dinv_ref, g1_ref):
    deg = deg_ref[...].sum(axis=0)
    dinv = jnp.where(deg > 0.5, lax.rsqrt(jnp.maximum(deg, 1.0)), 0.0)
    dinv_ref[...] = dinv
    g1_ref[...] = dinv * h1_ref[...]


def _mid_body(acc_ref, dinv_ref, b1_ref, g2_ref):
    dinv = dinv_ref[...]
    p = dinv * acc_ref[...].sum(axis=0) + b1_ref[...]
    g2_ref[...] = dinv * jnp.maximum(p, 0.0)


def _final_body(acc_ref, dinv_ref, w2_ref, b2_ref, o_ref):
    p2 = dinv_ref[...] * acc_ref[...].sum(axis=0)
    o_ref[...] = jnp.dot(p2, w2_ref[...],
                         preferred_element_type=jnp.float32) + b2_ref[...]


_RB = 1000


def kernel(t, x, edge_index, W1, b1, W2, b2):
    del t
    f32 = jnp.float32

    src = edge_index[0].astype(jnp.int32)
    dst = edge_index[1].astype(jnp.int32)
    padv = PADIDX + (jnp.arange(EPAD - E, dtype=jnp.int32) % 8)
    src_t = jnp.concatenate([src, padv]).reshape(NW, NCH, CHUNK)
    dst_t = jnp.concatenate([dst, padv]).reshape(NW, NCH, CHUNK)

    W1p = jnp.zeros((D, HP), f32).at[:, :H].set(W1)
    W2p = jnp.zeros((HP, D), f32).at[:H, :].set(W2)
    b1p = jnp.zeros((1, HP), f32).at[0, :H].set(b1)
    b2r = b2.reshape(1, D)

    real_row = (lax.broadcasted_iota(jnp.int32, (NPAD, HP), 0) < N)
    ones_table = jnp.where(real_row, 1.0, 0.0).astype(f32)
    zeros_init = jnp.zeros((NPAD, HP), f32)

    def two_part_init(part0):
        parts = [part0] + [zeros_init] * (NC - 1)
        return jnp.stack(parts)

    h1 = pl.pallas_call(
        _mm1_body,
        grid=(N // _RB,),
        in_specs=[pl.BlockSpec((_RB, D), lambda i: (i, 0)),
                  pl.BlockSpec((D, HP), lambda i: (0, 0))],
        out_specs=pl.BlockSpec((_RB, HP), lambda i: (i, 0)),
        out_shape=jax.ShapeDtypeStruct((N, HP), f32),
    )(x, W1p)
    h1p = jnp.zeros((NPAD, HP), f32).at[:N].set(h1)

    deg_parts = _sc_degree(ones_table, src_t, dst_t,
                           two_part_init(ones_table))

    dinv, g1 = pl.pallas_call(
        _prep1_body,
        out_shape=[jax.ShapeDtypeStruct((NPAD, HP), f32),
                   jax.ShapeDtypeStruct((NPAD, HP), f32)],
    )(deg_parts, h1p)

    acc1_parts = _sc_propagate(g1, src_t, dst_t, two_part_init(g1))

    g2 = pl.pallas_call(
        _mid_body,
        out_shape=jax.ShapeDtypeStruct((NPAD, HP), f32),
    )(acc1_parts, dinv, b1p)

    acc2_parts = _sc_propagate(g2, src_t, dst_t, two_part_init(g2))

    out = pl.pallas_call(
        _final_body,
        grid=(N // _RB,),
        in_specs=[pl.BlockSpec((NC, _RB, HP), lambda i: (0, i, 0)),
                  pl.BlockSpec((_RB, HP), lambda i: (i, 0)),
                  pl.BlockSpec((HP, D), lambda i: (0, 0)),
                  pl.BlockSpec((1, D), lambda i: (0, 0))],
        out_specs=pl.BlockSpec((_RB, D), lambda i: (i, 0)),
        out_shape=jax.ShapeDtypeStruct((N, D), f32),
    )(acc2_parts, dinv, W2p, b2r)
    return out


# speedup vs baseline: 38.7620x; 1.2442x over previous
import functools

import jax
import jax.numpy as jnp
from jax import lax
from jax.experimental import pallas as pl
from jax.experimental.pallas import tpu as pltpu
from jax.experimental.pallas import tpu_sc as plsc

N = 10000
D = 256
H = 5
E = 160000

HP = 8
NPAD = 10240
NS = 16
CHUNK = 128
EC = 10240
NCH = EC // CHUNK
EPAD = EC * NS
PADIDX = NPAD - 8
RPT = NPAD // NS
NV = RPT * HP // 16


def _rsqrt16(d):
    i = plsc.bitcast(d, jnp.int32)
    y = plsc.bitcast(0x5F3759DF - (i >> 1), jnp.float32)
    for _ in range(3):
        y = y * (1.5 - 0.5 * d * y * y)
    return y


def _fused_sc_body(h1_hbm, src_hbm, dst_hbm, ones_hbm, zeros_hbm, b1_hbm,
                   p2_hbm, g1_hbm, g2_hbm,
                   idx_s_v, idx_d_v, rows_v, gbuf, dinvbuf, abuf, zbuf,
                   ones_v, dummy_v, b1buf, acc_sh, gsem, ssem):
    sid = lax.axis_index("s")
    sl = pl.ds(sid * RPT, RPT)

    lane = lax.iota(jnp.int32, 16)
    cols = lane & 7
    rowpat = lane >> 3

    pltpu.sync_copy(src_hbm.at[sid], idx_s_v)
    pltpu.sync_copy(dst_hbm.at[sid], idx_d_v)
    pltpu.sync_copy(ones_hbm, ones_v)
    pltpu.sync_copy(zeros_hbm, zbuf)
    pltpu.sync_copy(b1_hbm, b1buf)
    pltpu.sync_copy(zbuf, acc_sh.at[sl])
    plsc.subcore_barrier()

    def _drain_scatters(n):
        def d(j, c):
            pltpu.make_async_copy(ones_hbm, dummy_v, ssem).wait()
            return c
        lax.fori_loop(0, n, d, 0)

    def _propagate(tab_hbm):
        def fire(j, c):
            pltpu.async_copy(tab_hbm.at[idx_s_v.at[j]], rows_v.at[j], gsem)
            return c
        lax.fori_loop(0, NCH, fire, 0)

        def scat(j, c):
            pltpu.make_async_copy(
                tab_hbm.at[pl.ds(0, CHUNK)], rows_v.at[j], gsem).wait()
            pltpu.async_copy(
                rows_v.at[j], acc_sh.at[idx_d_v.at[j]], ssem, add=True)
            return c
        lax.fori_loop(0, NCH, scat, 0)
        _drain_scatters(NCH)

    def deg_scat(j, c):
        pltpu.async_copy(ones_v, acc_sh.at[idx_d_v.at[j]], ssem, add=True)
        return c
    lax.fori_loop(0, NCH, deg_scat, 0)
    _drain_scatters(NCH)
    plsc.subcore_barrier()

    pltpu.sync_copy(acc_sh.at[sl], abuf)
    pltpu.sync_copy(zbuf, acc_sh.at[sl])
    pltpu.sync_copy(h1_hbm.at[sl], dinvbuf)

    def ew_a(i, c):
        rows = rowpat + 2 * i
        d = plsc.load_gather(abuf, [rows, cols]) + 1.0
        h = plsc.load_gather(dinvbuf, [rows, cols])
        y = _rsqrt16(d)
        plsc.store_scatter(dinvbuf, [rows, cols], y)
        plsc.store_scatter(gbuf, [rows, cols], y * h)
        return c
    lax.fori_loop(0, NV, ew_a, 0)
    pltpu.sync_copy(gbuf, g1_hbm.at[sl])
    plsc.subcore_barrier()

    _propagate(g1_hbm)
    plsc.subcore_barrier()

    pltpu.sync_copy(acc_sh.at[sl], abuf)
    pltpu.sync_copy(zbuf, acc_sh.at[sl])
    b1v = b1buf[...]

    def ew_b(i, c):
        rows = rowpat + 2 * i
        a = plsc.load_gather(abuf, [rows, cols])
        g = plsc.load_gather(gbuf, [rows, cols])
        y = plsc.load_gather(dinvbuf, [rows, cols])
        p = y * (a + g) + b1v
        plsc.store_scatter(gbuf, [rows, cols], y * jnp.maximum(p, 0.0))
        return c
    lax.fori_loop(0, NV, ew_b, 0)
    pltpu.sync_copy(gbuf, g2_hbm.at[sl])
    plsc.subcore_barrier()

    _propagate(g2_hbm)
    plsc.subcore_barrier()

    pltpu.sync_copy(acc_sh.at[sl], abuf)

    def ew_c(i, c):
        rows = rowpat + 2 * i
        a = plsc.load_gather(abuf, [rows, cols])
        g = plsc.load_gather(gbuf, [rows, cols])
        y = plsc.load_gather(dinvbuf, [rows, cols])
        plsc.store_scatter(abuf, [rows, cols], y * (a + g))
        return c
    lax.fori_loop(0, NV, ew_c, 0)
    pltpu.sync_copy(abuf, p2_hbm.at[sl])


_fused_sc = pl.kernel(
    _fused_sc_body,
    out_type=(jax.ShapeDtypeStruct((NPAD, HP), jnp.float32),
              jax.ShapeDtypeStruct((NPAD, HP), jnp.float32),
              jax.ShapeDtypeStruct((NPAD, HP), jnp.float32)),
    mesh=plsc.VectorSubcoreMesh(
        core_axis_name="c", subcore_axis_name="s",
        num_cores=1, num_subcores=NS),
    scratch_types=[
        pltpu.VMEM((NCH, CHUNK), jnp.int32),
        pltpu.VMEM((NCH, CHUNK), jnp.int32),
        pltpu.VMEM((NCH, CHUNK, HP), jnp.float32),
        pltpu.VMEM((RPT, HP), jnp.float32),
        pltpu.VMEM((RPT, HP), jnp.float32),
        pltpu.VMEM((RPT, HP), jnp.float32),
        pltpu.VMEM((RPT, HP), jnp.float32),
        pltpu.VMEM((CHUNK, HP), jnp.float32),
        pltpu.VMEM((CHUNK, HP), jnp.float32),
        pltpu.VMEM((16,), jnp.float32),
        pltpu.VMEM_SHARED((NPAD, HP), jnp.float32),
        pltpu.SemaphoreType.DMA,
        pltpu.SemaphoreType.DMA,
    ],
    compiler_params=pltpu.CompilerParams(
        use_tc_tiling_on_sc=False, needs_layout_passes=False),
)


def _mm1_body(x_ref, w_ref, o_ref):
    o_ref[...] = jnp.dot(x_ref[...], w_ref[...],
                         preferred_element_type=jnp.float32)


def _final_body(p2_ref, w2_ref, b2_ref, o_ref):
    o_ref[...] = jnp.dot(p2_ref[...], w2_ref[...],
                         preferred_element_type=jnp.float32) + b2_ref[...]


_RB = 1000


def kernel(t, x, edge_index, W1, b1, W2, b2):
    del t
    f32 = jnp.float32

    src = edge_index[0].astype(jnp.int32)
    dst = edge_index[1].astype(jnp.int32)
    padv = PADIDX + (jnp.arange(EPAD - E, dtype=jnp.int32) % 8)
    src_t = jnp.concatenate([src, padv]).reshape(NS, NCH, CHUNK)
    dst_t = jnp.concatenate([dst, padv]).reshape(NS, NCH, CHUNK)

    W1p = jnp.zeros((D, HP), f32).at[:, :H].set(W1)
    W2p = jnp.zeros((HP, D), f32).at[:H, :].set(W2)
    b1v = jnp.zeros((16,), f32).at[:H].set(b1).at[8:8 + H].set(b1)
    b2r = b2.reshape(1, D)

    ones_c = jnp.ones((CHUNK, HP), f32)
    zeros_r = jnp.zeros((RPT, HP), f32)

    h1 = pl.pallas_call(
        _mm1_body,
        grid=(N // _RB,),
        in_specs=[pl.BlockSpec((_RB, D), lambda i: (i, 0)),
                  pl.BlockSpec((D, HP), lambda i: (0, 0))],
        out_specs=pl.BlockSpec((_RB, HP), lambda i: (i, 0)),
        out_shape=jax.ShapeDtypeStruct((N, HP), f32),
    )(x, W1p)
    h1p = jnp.zeros((NPAD, HP), f32).at[:N].set(h1)

    p2, _, _ = _fused_sc(h1p, src_t, dst_t, ones_c, zeros_r, b1v)

    out = pl.pallas_call(
        _final_body,
        grid=(N // _RB,),
        in_specs=[pl.BlockSpec((_RB, HP), lambda i: (i, 0)),
                  pl.BlockSpec((HP, D), lambda i: (0, 0)),
                  pl.BlockSpec((1, D), lambda i: (0, 0))],
        out_specs=pl.BlockSpec((_RB, D), lambda i: (i, 0)),
        out_shape=jax.ShapeDtypeStruct((N, D), f32),
    )(p2, W2p, b2r)
    return out
